# baseline (speedup 1.0000x reference)
import functools

import jax
import jax.numpy as jnp
from jax import lax
from jax.experimental import pallas as pl
from jax.experimental.pallas import tpu as pltpu
from jax.experimental.pallas import tpu_sc as plsc

_NUM_MOODS = 1000
_D = 128
_B = 16384
_NC = 2
_NS = 16
_NW = _NC * _NS
_CHUNK = 128

_SC_N = 8192
_BPW = _SC_N // _NW
_NCHUNK = _BPW // _CHUNK

_mesh = plsc.VectorSubcoreMesh(core_axis_name="c", subcore_axis_name="s")


@functools.partial(
    pl.kernel,
    mesh=_mesh,
    out_type=jax.ShapeDtypeStruct((_SC_N, _D), jnp.float32),
    scratch_types=[
        pltpu.VMEM((_NCHUNK, _CHUNK), jnp.int32),
        pltpu.VMEM((_BPW, _D), jnp.float32),
        *([pltpu.SemaphoreType.DMA] * _NCHUNK),
    ],
)
def _sc_gather(idx_hbm, table_hbm, out_hbm, idx_v, rows_v, *g_sems):
    wid = lax.axis_index("s") * _NC + lax.axis_index("c")
    base = wid * _BPW
    pltpu.sync_copy(idx_hbm.at[wid], idx_v)
    gathers = [
        pltpu.async_copy(
            table_hbm.at[idx_v.at[j]],
            rows_v.at[pl.ds(j * _CHUNK, _CHUNK)],
            g_sems[j],
        )
        for j in range(_NCHUNK)
    ]
    for g in gathers:
        g.wait()
    pltpu.sync_copy(rows_v, out_hbm.at[pl.ds(base, _BPW)])


_V = 1024
_BB = 512


def _tc_body(idx_ref, hi_ref, lo_ref, out_ref):
    idx = idx_ref[0]
    iota_v = lax.broadcasted_iota(jnp.int32, (_BB, _V), 1)
    oh = jnp.where(idx == iota_v, 1.0, 0.0).astype(jnp.bfloat16)
    dn = (((1,), (0,)), ((), ()))
    acc = lax.dot_general(oh, hi_ref[...], dn, preferred_element_type=jnp.float32)
    acc = acc + lax.dot_general(oh, lo_ref[...], dn, preferred_element_type=jnp.float32)
    out_ref[...] = acc


def _tc_gather(idx, hi, lo):
    n = idx.shape[0]
    nsteps = n // _BB
    idx2 = idx.reshape(nsteps, _BB, 1)
    return pl.pallas_call(
        _tc_body,
        grid=(nsteps,),
        in_specs=[
            pl.BlockSpec((1, _BB, 1), lambda i: (i, 0, 0)),
            pl.BlockSpec((_V, _D), lambda i: (0, 0)),
            pl.BlockSpec((_V, _D), lambda i: (0, 0)),
        ],
        out_specs=pl.BlockSpec((_BB, _D), lambda i: (i, 0)),
        out_shape=jax.ShapeDtypeStruct((n, _D), jnp.float32),
    )(idx2, hi, lo)


def kernel(mood_indices, mood_embedding_weight):
    idx = mood_indices.astype(jnp.int32)
    sc_idx = idx[:_SC_N].reshape(_NW, _NCHUNK, _CHUNK)
    sc_out = _sc_gather(sc_idx, mood_embedding_weight)

    hi_f32 = lax.reduce_precision(mood_embedding_weight, 8, 7)
    hi = hi_f32.astype(jnp.bfloat16)
    lo = (mood_embedding_weight - hi_f32).astype(jnp.bfloat16)
    hi = jnp.pad(hi, ((0, _V - _NUM_MOODS), (0, 0)))
    lo = jnp.pad(lo, ((0, _V - _NUM_MOODS), (0, 0)))
    tc_out = _tc_gather(idx[_SC_N:], hi, lo)
    return jnp.concatenate([sc_out, tc_out], axis=0)


# speedup vs baseline: 1.8977x; 1.8977x over previous
import functools

import jax
import jax.numpy as jnp
from jax import lax
from jax.experimental import pallas as pl
from jax.experimental.pallas import tpu as pltpu
from jax.experimental.pallas import tpu_sc as plsc

_NUM_MOODS = 1000
_D = 128
_B = 16384
_NC = 2
_NS = 16
_NW = _NC * _NS
_BPW = _B // _NW
_CHUNK = 128
_NCHUNK = _BPW // _CHUNK

_mesh = plsc.VectorSubcoreMesh(core_axis_name="c", subcore_axis_name="s")


@functools.partial(
    pl.kernel,
    mesh=_mesh,
    out_type=jax.ShapeDtypeStruct((_B, _D), jnp.float32),
    scratch_types=[
        pltpu.VMEM((_NCHUNK, _CHUNK), jnp.int32),
        pltpu.VMEM((_BPW, _D), jnp.float32),
        *([pltpu.SemaphoreType.DMA] * _NCHUNK),
    ],
)
def _sc_gather(idx_hbm, table_hbm, out_hbm, idx_v, rows_v, *g_sems):
    wid = lax.axis_index("s") * _NC + lax.axis_index("c")
    base = wid * _BPW
    pltpu.sync_copy(idx_hbm.at[wid], idx_v)
    gathers = [
        pltpu.async_copy(
            table_hbm.at[idx_v.at[j]],
            rows_v.at[pl.ds(j * _CHUNK, _CHUNK)],
            g_sems[j],
        )
        for j in range(_NCHUNK)
    ]
    for g in gathers:
        g.wait()
    pltpu.sync_copy(rows_v, out_hbm.at[pl.ds(base, _BPW)])


def kernel(mood_indices, mood_embedding_weight):
    idx = mood_indices.astype(jnp.int32).reshape(_NW, _NCHUNK, _CHUNK)
    return _sc_gather(idx, mood_embedding_weight)
